# Initial kernel scaffold; baseline (speedup 1.0000x reference)
import jax
import jax.numpy as jnp
from jax.experimental import pallas as pl


def kernel(x, old_W, new_W):
    raise NotImplementedError("write your pallas kernel here")


# speedup vs baseline: 6.1314x; 6.1314x over previous
import functools

import jax
import jax.numpy as jnp
from jax import lax
from jax.experimental import pallas as pl
from jax.experimental.pallas import tpu as pltpu
from jax.experimental.pallas import tpu_sc as plsc

NUM_OLD = 100000
NUM_TOTAL = 1000000
DIM = 32

NC = 2
NS = 16
L = 16
NW = NC * NS
N = 4096 * 200
CHUNK = N // NW
B = 128
NVR = CHUNK // L
G = 8

_MESH = plsc.VectorSubcoreMesh(
    core_axis_name="c", subcore_axis_name="s", num_cores=NC, num_subcores=NS
)

_PARAMS = pltpu.CompilerParams(
    needs_layout_passes=False, use_tc_tiling_on_sc=False
)


def _wid():
    return lax.axis_index("c") * NS + lax.axis_index("s")


@functools.partial(
    pl.kernel,
    out_type=jax.ShapeDtypeStruct((NW, L), jnp.int32),
    mesh=_MESH,
    scratch_types=[
        pltpu.VMEM((CHUNK,), jnp.int32),
        pltpu.VMEM((L,), jnp.int32),
    ],
    compiler_params=_PARAMS,
)
def _count_kernel(x_hbm, counts_hbm, xv, cnt_v):
    wid = _wid()
    base = pl.multiple_of(wid * CHUNK, CHUNK)
    pltpu.sync_copy(x_hbm.at[pl.ds(base, CHUNK)], xv)

    def step(i, acc):
        v = xv[pl.ds(i * L, L)]
        return acc + jnp.where(v < NUM_OLD, 1, 0).astype(jnp.int32)

    acc = lax.fori_loop(0, NVR, step, jnp.zeros((L,), jnp.int32))
    n_old = jnp.sum(acc)
    cnt_v[...] = jnp.full((L,), n_old, jnp.int32)
    pltpu.sync_copy(cnt_v, counts_hbm.at[wid])


@functools.partial(
    pl.kernel,
    out_type=jax.ShapeDtypeStruct((N, DIM), jnp.float32),
    mesh=_MESH,
    scratch_types=[
        pltpu.VMEM((CHUNK,), jnp.int32),
        pltpu.VMEM((CHUNK,), jnp.int32),
        pltpu.VMEM((CHUNK,), jnp.int32),
        pltpu.VMEM((NW, L), jnp.int32),
        pltpu.VMEM((G, B, DIM), jnp.float32),
        pltpu.SemaphoreType.DMA,
        pltpu.SemaphoreType.DMA,
    ],
    compiler_params=_PARAMS,
)
def _place_kernel(x_hbm, counts_hbm, old_w_hbm, new_w_hbm, out_hbm,
                  xv, oi_v, ni_v, cnts_v, rows_v, gsem, ssem):
    wid = _wid()
    base = pl.multiple_of(wid * CHUNK, CHUNK)

    pltpu.sync_copy(counts_hbm, cnts_v)
    pltpu.sync_copy(x_hbm.at[pl.ds(base, CHUNK)], xv)

    def pfx(w, carry):
        o_base, tot = carry
        n = jnp.max(cnts_v[w, :])
        return (o_base + jnp.where(w < wid, n, 0), tot + n)

    old_base, total_old = lax.fori_loop(
        0, NW, pfx, (jnp.int32(0), jnp.int32(0))
    )
    new_base = total_old + base - old_base

    iota = lax.iota(jnp.int32, L)

    def comp(i, carry):
        c_old, c_new = carry
        v = xv[pl.ds(i * L, L)]
        m = v < NUM_OLD
        mi = jnp.where(m, 1, 0).astype(jnp.int32)
        cs = plsc.cumsum(mi)
        s = plsc.all_reduce_population_count(m)
        plsc.store_scatter(oi_v, [c_old + cs - 1], v, mask=m)
        plsc.store_scatter(
            ni_v, [c_new + iota - cs], v - NUM_OLD, mask=jnp.logical_not(m)
        )
        return (c_old + s, c_new + (L - s))

    zeros = jnp.zeros((L,), jnp.int32)
    c_old_v, _ = lax.fori_loop(0, NVR, comp, (zeros, zeros))
    n_old = jnp.max(c_old_v)
    n_new = CHUNK - n_old

    def run_table(idx_ref, table_ref, count, out_base):
        nfull = count // B
        ngroups = nfull // G

        def do_group(gi, _):
            b0 = pl.multiple_of(gi * G * B, G * B)
            gd = [
                pltpu.async_copy(
                    table_ref.at[idx_ref.at[pl.ds(b0 + k * B, B)]],
                    rows_v.at[k], gsem,
                )
                for k in range(G)
            ]
            for d in gd:
                d.wait()
            sd = [
                pltpu.async_copy(
                    rows_v.at[k],
                    out_hbm.at[pl.ds(out_base + b0 + k * B, B)],
                    ssem,
                )
                for k in range(G)
            ]
            for d in sd:
                d.wait()
            return 0

        lax.fori_loop(0, ngroups, do_group, 0)

        def do_blk(b, _):
            k0 = pl.multiple_of(b * B, B)
            pltpu.async_copy(
                table_ref.at[idx_ref.at[pl.ds(k0, B)]], rows_v.at[0], gsem
            ).wait()
            pltpu.async_copy(
                rows_v.at[0], out_hbm.at[pl.ds(out_base + b * B, B)], ssem
            ).wait()
            return 0

        lax.fori_loop(ngroups * G, nfull, do_blk, 0)

        t0 = nfull * B
        t = count - t0

        @pl.when(t > 0)
        def _():
            def san(j, _):
                off = t0 + j * L
                k_vec = off + iota
                raw = idx_ref[pl.ds(off, L)]
                idx_ref[pl.ds(off, L)] = jnp.where(k_vec < count, raw, 0)
                return 0

            lax.fori_loop(0, B // L, san, 0)
            pltpu.async_copy(
                table_ref.at[idx_ref.at[pl.ds(pl.multiple_of(t0, B), B)]],
                rows_v.at[0], gsem,
            ).wait()
            for sz in (64, 32, 16, 8, 4, 2, 1):
                off_rel = jnp.bitwise_and(t, ~(2 * sz - 1))

                @pl.when(jnp.bitwise_and(t, sz) != 0)
                def _(off_rel=off_rel, sz=sz):
                    pltpu.sync_copy(
                        rows_v.at[0].at[pl.ds(off_rel, sz)],
                        out_hbm.at[pl.ds(out_base + t0 + off_rel, sz)],
                    )

    run_table(oi_v, old_w_hbm, n_old, old_base)
    run_table(ni_v, new_w_hbm, n_new, new_base)


def kernel(x, old_W, new_W):
    xf = x.reshape(-1)
    counts = _count_kernel(xf)
    return _place_kernel(xf, counts, old_W, new_W)
